# baseline (speedup 1.0000x reference)
import functools

import jax
import jax.numpy as jnp
from jax.experimental import pallas as pl
from jax.experimental.pallas import tpu as pltpu

_CHUNK = 2560
_RING = 8


def _mlp_chunk(x, rbf, wrbf, wa, b1, w2, b2, wout):
    def silu(v):
        return 0.5 * v * (1.0 + jnp.tanh(0.5 * v))

    h = jnp.dot(rbf, wrbf, preferred_element_type=jnp.float32) * x
    z1t = jax.lax.dot_general(wa, h, (((0,), (1,)), ((), ())),
                              preferred_element_type=jnp.float32)
    ot = silu(z1t + b1)
    z2t = jax.lax.dot_general(w2, ot, (((0,), (0,)), ((), ())),
                              preferred_element_type=jnp.float32)
    ot = silu(z2t + b2)
    return jax.lax.dot_general(wout, ot, (((0,), (0,)), ((), ())),
                               preferred_element_type=jnp.float32)


def _kernel(x_hbm, rbf_hbm, wrbf_ref, wup_ref, w1_ref, b1_ref, w2_ref,
            b2_ref, wout_ref, o_ref, xbuf, rbfbuf, xsem, rsem, *, n_chunks):
    wa = jnp.dot(wup_ref[...], w1_ref[...],
                 preferred_element_type=jnp.float32)

    def x_copy(c, slot):
        return pltpu.make_async_copy(
            x_hbm.at[pl.ds(c * _CHUNK, _CHUNK), :], xbuf.at[slot],
            xsem.at[slot])

    def rbf_copy(c, slot):
        return pltpu.make_async_copy(
            rbf_hbm.at[pl.ds(c * _CHUNK, _CHUNK), :], rbfbuf.at[slot],
            rsem.at[slot])

    for w in range(_RING):
        x_copy(w, w).start()
        rbf_copy(w, w).start()

    def body(c, carry):
        slot = jax.lax.rem(c, _RING)
        x_copy(c, slot).wait()
        rbf_copy(c, slot).wait()
        outt = _mlp_chunk(xbuf[slot], rbfbuf[slot], wrbf_ref[...], wa,
                          b1_ref[...], w2_ref[...], b2_ref[...],
                          wout_ref[...])
        o_ref[pl.ds(c * (_CHUNK // 128), _CHUNK // 128), :, :] = (
            outt.reshape(_CHUNK // 128, 1, 128))

        @pl.when(c + _RING < n_chunks)
        def _():
            x_copy(c + _RING, slot).start()
            rbf_copy(c + _RING, slot).start()

        return carry

    jax.lax.fori_loop(0, n_chunks, body, 0)


def kernel(x, rbf, i, num_nodes, W_rbf, W_up, W1, b1, W2, b2, W_out):
    del i, num_nodes
    E, H = x.shape
    R = rbf.shape[1]
    D = W_up.shape[1]
    b1 = b1.reshape(D, 1)
    b2 = b2.reshape(D, 1)
    n_chunks = E // _CHUNK

    rep = lambda shape: pl.BlockSpec(shape, lambda: (0, 0))
    hbm = pl.BlockSpec(memory_space=pltpu.MemorySpace.HBM)

    out2 = pl.pallas_call(
        functools.partial(_kernel, n_chunks=n_chunks),
        grid=(),
        in_specs=[
            hbm,
            hbm,
            rep((R, H)),
            rep((H, D)),
            rep((D, D)),
            rep((D, 1)),
            rep((D, D)),
            rep((D, 1)),
            rep((D, 1)),
        ],
        out_specs=pl.BlockSpec((E // 128, 1, 128), lambda: (0, 0, 0)),
        out_shape=jax.ShapeDtypeStruct((E // 128, 1, 128), jnp.float32),
        scratch_shapes=[
            pltpu.VMEM((_RING, _CHUNK, H), jnp.float32),
            pltpu.VMEM((_RING, _CHUNK, R), jnp.float32),
            pltpu.SemaphoreType.DMA((_RING,)),
            pltpu.SemaphoreType.DMA((_RING,)),
        ],
    )(x, rbf, W_rbf, W_up, W1, b1, W2, b2, W_out)
    return out2.reshape(E, 1)


# speedup vs baseline: 1.0093x; 1.0093x over previous
import jax
import jax.numpy as jnp
from jax.experimental import pallas as pl
from jax.experimental.pallas import tpu as pltpu

_CHUNK = 3200
_RING = 6
_NCORES = 2


def _mlp_chunk(x, rbf, wrbf, wa, b1, w2, b2, wout):
    def silu(v):
        return 0.5 * v * (1.0 + jnp.tanh(0.5 * v))

    h = jnp.dot(rbf, wrbf, preferred_element_type=jnp.float32) * x
    z1t = jax.lax.dot_general(wa, h, (((0,), (1,)), ((), ())),
                              preferred_element_type=jnp.float32)
    ot = silu(z1t + b1)
    z2t = jax.lax.dot_general(w2, ot, (((0,), (0,)), ((), ())),
                              preferred_element_type=jnp.float32)
    ot = silu(z2t + b2)
    return jax.lax.dot_general(wout, ot, (((0,), (0,)), ((), ())),
                               preferred_element_type=jnp.float32)


def _make_body(n_chunks_per_core, pack_rows):
    def body(x_hbm, rbf_hbm, wrbf_h, wup_h, w1_h, b1_h, w2_h, b2_h, wout_h,
             o_hbm, xbuf, rbfbuf, wrbf_v, wup_v, w1_v, b1_v, w2_v, b2_v,
             wout_v, ovmem, xsem, rsem, wsem, osem):
        core = jax.lax.axis_index("core")

        for src, dst in ((wrbf_h, wrbf_v), (wup_h, wup_v), (w1_h, w1_v),
                         (b1_h, b1_v), (w2_h, w2_v), (b2_h, b2_v),
                         (wout_h, wout_v)):
            cp = pltpu.make_async_copy(src, dst, wsem)
            cp.start()
            cp.wait()
        wa = jnp.dot(wup_v[...], w1_v[...],
                     preferred_element_type=jnp.float32)

        base = core * n_chunks_per_core

        def x_copy(c, slot):
            return pltpu.make_async_copy(
                x_hbm.at[pl.ds(c * _CHUNK, _CHUNK), :], xbuf.at[slot],
                xsem.at[slot])

        def rbf_copy(c, slot):
            return pltpu.make_async_copy(
                rbf_hbm.at[pl.ds(c * _CHUNK, _CHUNK), :], rbfbuf.at[slot],
                rsem.at[slot])

        for w in range(_RING):
            x_copy(base + w, w).start()
            rbf_copy(base + w, w).start()

        def loop(j, carry):
            c = base + j
            slot = jax.lax.rem(j, _RING)
            x_copy(c, slot).wait()
            rbf_copy(c, slot).wait()
            outt = _mlp_chunk(xbuf[slot], rbfbuf[slot], wrbf_v[...], wa,
                              b1_v[...], w2_v[...], b2_v[...], wout_v[...])
            ovmem[pl.ds(j * pack_rows, pack_rows), :, :] = (
                outt.reshape(pack_rows, 1, 128))

            @pl.when(j + _RING < n_chunks_per_core)
            def _():
                x_copy(c + _RING, slot).start()
                rbf_copy(c + _RING, slot).start()

            return carry

        jax.lax.fori_loop(0, n_chunks_per_core, loop, 0)

        half = n_chunks_per_core * pack_rows
        out_cp = pltpu.make_async_copy(
            ovmem, o_hbm.at[pl.ds(core * half, half), :, :], osem)
        out_cp.start()
        out_cp.wait()

    return body


def kernel(x, rbf, i, num_nodes, W_rbf, W_up, W1, b1, W2, b2, W_out):
    del i, num_nodes
    E, H = x.shape
    R = rbf.shape[1]
    D = W_up.shape[1]
    b1 = b1.reshape(D, 1)
    b2 = b2.reshape(D, 1)
    n_per_core = E // _CHUNK // _NCORES
    pack_rows = _CHUNK // 128

    mesh = pltpu.create_tensorcore_mesh("core", num_cores=_NCORES)
    f32 = jnp.float32
    out2 = pl.kernel(
        _make_body(n_per_core, pack_rows),
        out_type=jax.ShapeDtypeStruct((E // 128, 1, 128), f32),
        mesh=mesh,
        scratch_types=[
            pltpu.VMEM((_RING, _CHUNK, H), f32),
            pltpu.VMEM((_RING, _CHUNK, R), f32),
            pltpu.VMEM((R, H), f32),
            pltpu.VMEM((H, D), f32),
            pltpu.VMEM((D, D), f32),
            pltpu.VMEM((D, 1), f32),
            pltpu.VMEM((D, D), f32),
            pltpu.VMEM((D, 1), f32),
            pltpu.VMEM((D, 1), f32),
            pltpu.VMEM((n_per_core * pack_rows, 1, 128), f32),
            pltpu.SemaphoreType.DMA((_RING,)),
            pltpu.SemaphoreType.DMA((_RING,)),
            pltpu.SemaphoreType.DMA,
            pltpu.SemaphoreType.DMA,
        ],
    )(x, rbf, W_rbf, W_up, W1, b1, W2, b2, W_out)
    return out2.reshape(E, 1)


# speedup vs baseline: 1.0962x; 1.0861x over previous
import jax
import jax.numpy as jnp
from jax.experimental import pallas as pl
from jax.experimental.pallas import tpu as pltpu

_BLOCK = 16000


def _mlp_block(x_ref, rbf_ref, wrbf_ref, wup_ref, w1_ref, b1_ref, w2_ref,
               b2_ref, wout_ref, o_ref):
    def silu(v):
        return 0.5 * v * (1.0 + jnp.tanh(0.5 * v))

    wa = jnp.dot(wup_ref[...], w1_ref[...],
                 preferred_element_type=jnp.float32)
    h = jnp.dot(rbf_ref[...], wrbf_ref[...],
                preferred_element_type=jnp.float32) * x_ref[...]
    z1t = jax.lax.dot_general(wa, h, (((0,), (1,)), ((), ())),
                              preferred_element_type=jnp.float32)
    ot = silu(z1t + b1_ref[...])
    z2t = jax.lax.dot_general(w2_ref[...], ot, (((0,), (0,)), ((), ())),
                              preferred_element_type=jnp.float32)
    ot = silu(z2t + b2_ref[...])
    outt = jax.lax.dot_general(wout_ref[...], ot, (((0,), (0,)), ((), ())),
                               preferred_element_type=jnp.float32)
    o_ref[...] = outt.reshape(o_ref.shape)


def kernel(x, rbf, i, num_nodes, W_rbf, W_up, W1, b1, W2, b2, W_out):
    del i, num_nodes
    E, H = x.shape
    R = rbf.shape[1]
    D = W_up.shape[1]
    b1 = b1.reshape(D, 1)
    b2 = b2.reshape(D, 1)

    grid = (E // _BLOCK,)
    row_spec = lambda shape: pl.BlockSpec(shape, lambda m: (m, 0))
    rep_spec = lambda shape: pl.BlockSpec(shape, lambda m: (0, 0))

    out2 = pl.pallas_call(
        _mlp_block,
        grid=grid,
        in_specs=[
            row_spec((_BLOCK, H)),
            row_spec((_BLOCK, R)),
            rep_spec((R, H)),
            rep_spec((H, D)),
            rep_spec((D, D)),
            rep_spec((D, 1)),
            rep_spec((D, D)),
            rep_spec((D, 1)),
            rep_spec((D, 1)),
        ],
        out_specs=pl.BlockSpec((_BLOCK // 128, 1, 128), lambda m: (m, 0, 0)),
        out_shape=jax.ShapeDtypeStruct((E // 128, 1, 128), jnp.float32),
        compiler_params=pltpu.CompilerParams(
            dimension_semantics=("parallel",)),
    )(x, rbf, W_rbf, W_up, W1, b1, W2, b2, W_out)
    return out2.reshape(E, 1)
